# baseline (speedup 1.0000x reference)
import functools

import jax
import jax.numpy as jnp
from jax import lax
from jax.experimental import pallas as pl
from jax.experimental.pallas import tpu as pltpu
from jax.experimental.pallas import tpu_sc as plsc

N = 10000
E = 160000
IN_DIM = 256
HID = 512
OUT_DIM = 128
NUM_LAYERS = 4

N_PAD = 10240
TR = 512
GRID = N_PAD // TR

NQ = 4
QD = HID // NQ
NT = 16
CH = 80
RPT = N_PAD // NT

CH2 = 128
E_PAD = 2 * NT * CH2 * 40
EPT = E_PAD // NT
NCH2 = EPT // CH2
GCH2 = 2 * CH2
NGRP2 = NCH2 // 2
CNT_W = 128
EPT_C = E // (2 * NT)
NCHP = EPT_C // CH + 1


def _count_body(dst2_hbm, cnt_hbm, dst_all, ones_v, acc_sh, ssem):
    c = lax.axis_index("c")
    s = lax.axis_index("s")
    w = c * NT + s
    zeros16 = jnp.zeros((16,), jnp.float32)

    pltpu.sync_copy(dst2_hbm.at[w], dst_all)

    def _zero_row(i, _):
        for j in range(CNT_W // 16):
            ones_v[i, pl.ds(16 * j, 16)] = zeros16
        return 0

    lax.fori_loop(0, CH, _zero_row, 0)
    for k in range(RPT // CH):
        pltpu.sync_copy(ones_v, acc_sh.at[pl.ds(s * RPT + k * CH, CH)])

    def _one_row(i, _):
        for j in range(CNT_W // 16):
            ones_v[i, pl.ds(16 * j, 16)] = zeros16 + 1.0
        return 0

    lax.fori_loop(0, CH, _one_row, 0)
    plsc.subcore_barrier()

    GK = 2

    def _grp(g, _):
        for b in range(GK):
            pltpu.async_copy(ones_v, acc_sh.at[dst_all.at[GK * g + b]],
                             ssem, add=True)
        for b in range(GK):
            pltpu.make_async_copy(ones_v, acc_sh.at[pl.ds(0, CH)],
                                  ssem).wait()
        return 0

    lax.fori_loop(0, NCHP // GK, _grp, 0)
    pltpu.async_copy(ones_v, acc_sh.at[dst_all.at[NCHP - 1]], ssem, add=True)
    pltpu.make_async_copy(ones_v, acc_sh.at[pl.ds(0, CH)], ssem).wait()
    plsc.subcore_barrier()

    pltpu.sync_copy(acc_sh.at[pl.ds(s * RPT, RPT)],
                    cnt_hbm.at[pl.ds(c * N_PAD + s * RPT, RPT)])


def _agg_body(hn_hbm, src_hbm, dst4_hbm, out_hbm, srcA, srcB, dstA, dstB,
              buf0, buf1, iA, jA, iB, jB, sem0, sem1, acc_sh):
    c = lax.axis_index("c")
    s = lax.axis_index("s")
    zeros16 = jnp.zeros((16,), jnp.float32)

    def _load(g, srcb, dstb, isem, jsem):
        g2 = jnp.where(g < NGRP2, g, 0)
        pltpu.async_copy(src_hbm.at[pl.ds(s * EPT + g2 * GCH2, GCH2)],
                         srcb, isem)
        pltpu.async_copy(dst4_hbm.at[s * NGRP2 + g2], dstb, jsem)

    def _load_wait(srcb, dstb, isem, jsem):
        pltpu.make_async_copy(src_hbm.at[pl.ds(0, GCH2)], srcb, isem).wait()
        pltpu.make_async_copy(dst4_hbm.at[0], dstb, jsem).wait()

    def _gwait(buf, sem):
        pltpu.make_async_copy(hn_hbm.at[pl.ds(0, CH2)], buf, sem).wait()

    for qi in range(2):
        qstart = (2 * c + qi) * N_PAD
        win = hn_hbm.at[pl.ds(qstart, N_PAD)]

        def _gather(srcb, half, buf, sem):
            pltpu.async_copy(
                win.at[srcb.at[pl.ds(half * CH2, CH2)]], buf, sem)

        def _zero_row(i, _):
            for j in range(QD // 16):
                buf0[i, pl.ds(16 * j, 16)] = zeros16
            return 0

        lax.fori_loop(0, CH2, _zero_row, 0)
        for k in range(RPT // CH2):
            pltpu.sync_copy(buf0, acc_sh.at[pl.ds(s * RPT + k * CH2, CH2)])
        plsc.subcore_barrier()

        _load(0, srcA, dstA, iA, jA)
        _load_wait(srcA, dstA, iA, jA)
        _gather(srcA, 0, buf0, sem0)

        def _iter(u, _):
            _load(2 * u + 1, srcB, dstB, iB, jB)
            _gather(srcA, 1, buf1, sem1)
            _gwait(buf0, sem0)
            pltpu.sync_copy(buf0, acc_sh.at[dstA.at[0]], add=True)
            _load_wait(srcB, dstB, iB, jB)
            _gather(srcB, 0, buf0, sem0)
            _gwait(buf1, sem1)
            pltpu.sync_copy(buf1, acc_sh.at[dstA.at[1]], add=True)
            _load(2 * u + 2, srcA, dstA, iA, jA)
            _gather(srcB, 1, buf1, sem1)
            _gwait(buf0, sem0)
            pltpu.sync_copy(buf0, acc_sh.at[dstB.at[0]], add=True)
            _load_wait(srcA, dstA, iA, jA)
            _gather(srcA, 0, buf0, sem0)
            _gwait(buf1, sem1)
            pltpu.sync_copy(buf1, acc_sh.at[dstB.at[1]], add=True)
            return 0

        lax.fori_loop(0, NGRP2 // 2, _iter, 0)
        _gwait(buf0, sem0)
        plsc.subcore_barrier()

        pltpu.sync_copy(acc_sh.at[pl.ds(s * RPT, RPT)],
                        out_hbm.at[pl.ds((2 * c + qi) * N_PAD + s * RPT, RPT)])


@functools.cache
def _sc_kernels():
    mesh = plsc.VectorSubcoreMesh(core_axis_name="c", subcore_axis_name="s")
    count_kernel = functools.partial(
        pl.kernel,
        out_type=jax.ShapeDtypeStruct((2 * N_PAD, CNT_W), jnp.float32),
        mesh=mesh,
        scratch_types=[
            pltpu.VMEM((NCHP, CH), jnp.int32),
            pltpu.VMEM((CH, CNT_W), jnp.float32),
            pltpu.VMEM_SHARED((N_PAD, CNT_W), jnp.float32),
            pltpu.SemaphoreType.DMA,
        ],
    )(_count_body)
    agg_kernel = functools.partial(
        pl.kernel,
        out_type=jax.ShapeDtypeStruct((NQ * N_PAD, QD), jnp.float32),
        mesh=mesh,
        scratch_types=(
            [
                pltpu.VMEM((GCH2,), jnp.int32),
                pltpu.VMEM((GCH2,), jnp.int32),
                pltpu.VMEM((2, CH2), jnp.int32),
                pltpu.VMEM((2, CH2), jnp.int32),
                pltpu.VMEM((CH2, QD), jnp.float32),
                pltpu.VMEM((CH2, QD), jnp.float32),
            ]
            + [pltpu.SemaphoreType.DMA for _ in range(6)]
            + [pltpu.VMEM_SHARED((N_PAD, QD), jnp.float32)]
        ),
    )(_agg_body)
    return count_kernel, agg_kernel


def _ln_self(h, g, bt, wr, br):
    mu = jnp.mean(h, axis=1, keepdims=True)
    var = jnp.mean((h - mu) ** 2, axis=1, keepdims=True)
    hn = (h - mu) * lax.rsqrt(var + 1e-5) * g + bt
    self_t = jnp.dot(hn, wr, preferred_element_type=jnp.float32) + br
    return hn.reshape(TR, NQ, QD).transpose(1, 0, 2), self_t


def _mix(agg4, cnt2, wl, bl, self_t, h, jk):
    agg = agg4.transpose(1, 0, 2).reshape(TR, HID)
    scale = 1.0 / jnp.maximum(cnt2[0, :, :1] + cnt2[1, :, :1], 1.0)
    z = (jnp.dot(agg * scale, wl, preferred_element_type=jnp.float32)
         + bl + self_t)
    hnew = jnp.maximum(h + z, 0.0)
    return hnew, jnp.maximum(jk, hnew)


def _in_body(x_ref, win_ref, bin_ref, g_ref, bt_ref, wr_ref, br_ref,
             h_ref, hn_ref, self_ref):
    h = (jnp.dot(x_ref[...], win_ref[...], preferred_element_type=jnp.float32)
         + bin_ref[...])
    h_ref[...] = h
    hn_ref[...], self_ref[...] = _ln_self(h, g_ref[...], bt_ref[...],
                                          wr_ref[...], br_ref[...])


def _layer_body(agg_ref, cnt_ref, wl_ref, bl_ref, self_ref, h_ref, jk_ref,
                g_ref, bt_ref, wr_ref, br_ref,
                ho_ref, jko_ref, hn_ref, selfo_ref):
    hnew, jknew = _mix(agg_ref[...], cnt_ref[...], wl_ref[...], bl_ref[...],
                       self_ref[...], h_ref[...], jk_ref[...])
    ho_ref[...] = hnew
    jko_ref[...] = jknew
    hn_ref[...], selfo_ref[...] = _ln_self(hnew, g_ref[...], bt_ref[...],
                                           wr_ref[...], br_ref[...])


def _fin_body(agg_ref, cnt_ref, wl_ref, bl_ref, self_ref, h_ref, jk_ref,
              wout_ref, bout_ref, o_ref):
    _, jknew = _mix(agg_ref[...], cnt_ref[...], wl_ref[...], bl_ref[...],
                    self_ref[...], h_ref[...], jk_ref[...])
    o_ref[...] = (jnp.dot(jknew, wout_ref[...],
                          preferred_element_type=jnp.float32) + bout_ref[...])


def _row_spec(w):
    return pl.BlockSpec((TR, w), lambda i: (i, 0))


def _full_spec(shape):
    return pl.BlockSpec(shape, lambda i: tuple(0 for _ in shape))


_agg_spec = pl.BlockSpec((NQ, TR, QD), lambda i: (0, i, 0))
_cnt_spec = pl.BlockSpec((2, TR, CNT_W), lambda i: (0, i, 0))
_hn_out = jax.ShapeDtypeStruct((NQ, N_PAD, QD), jnp.float32)
_row_out = jax.ShapeDtypeStruct((N_PAD, HID), jnp.float32)

_in_call = pl.pallas_call(
    _in_body,
    grid=(GRID,),
    in_specs=[_row_spec(IN_DIM), _full_spec((IN_DIM, HID)),
              _full_spec((1, HID)), _full_spec((1, HID)), _full_spec((1, HID)),
              _full_spec((HID, HID)), _full_spec((1, HID))],
    out_specs=[_row_spec(HID), _agg_spec, _row_spec(HID)],
    out_shape=[_row_out, _hn_out, _row_out],
)

_layer_call = pl.pallas_call(
    _layer_body,
    grid=(GRID,),
    in_specs=[_agg_spec, _cnt_spec, _full_spec((HID, HID)),
              _full_spec((1, HID)), _row_spec(HID), _row_spec(HID),
              _row_spec(HID), _full_spec((1, HID)), _full_spec((1, HID)),
              _full_spec((HID, HID)), _full_spec((1, HID))],
    out_specs=[_row_spec(HID), _row_spec(HID), _agg_spec, _row_spec(HID)],
    out_shape=[_row_out, _row_out, _hn_out, _row_out],
)

_fin_call = pl.pallas_call(
    _fin_body,
    grid=(GRID,),
    in_specs=[_agg_spec, _cnt_spec, _full_spec((HID, HID)),
              _full_spec((1, HID)), _row_spec(HID), _row_spec(HID),
              _row_spec(HID), _full_spec((HID, OUT_DIM)),
              _full_spec((1, OUT_DIM))],
    out_specs=_row_spec(OUT_DIM),
    out_shape=jax.ShapeDtypeStruct((N_PAD, OUT_DIM), jnp.float32),
)


def kernel(x, edge_index, params):
    src = edge_index[0].astype(jnp.int32)
    dst = edge_index[1].astype(jnp.int32)
    src_p = jnp.pad(src, (0, E_PAD - E))
    dst4 = jnp.pad(dst, (0, E_PAD - E),
                   constant_values=N_PAD - 1).reshape(NT * NGRP2, 2, CH2)
    dstc = jnp.pad(dst.reshape(2 * NT, EPT_C),
                   ((0, 0), (0, NCHP * CH - EPT_C)),
                   constant_values=N_PAD - 1).reshape(2 * NT, NCHP, CH)
    x_p = jnp.pad(x, ((0, N_PAD - N), (0, 0)))

    p = params
    count_kernel, agg_kernel = _sc_kernels()
    cnt = count_kernel(dstc).reshape(2, N_PAD, CNT_W)

    def b1(v):
        return v.reshape(1, -1)

    h, hn4, self_t = _in_call(x_p, p["Win"], b1(p["bin"]), b1(p["ln_g"][0]),
                              b1(p["ln_b"][0]), p["Wr"][0], b1(p["br"][0]))

    jk = jnp.zeros((N_PAD, HID), jnp.float32)
    for i in range(NUM_LAYERS - 1):
        agg = agg_kernel(hn4.reshape(NQ * N_PAD, QD), src_p, dst4)
        h, jk, hn4, self_t = _layer_call(
            agg.reshape(NQ, N_PAD, QD), cnt, p["Wl"][i], b1(p["bl"][i]),
            self_t, h, jk, b1(p["ln_g"][i + 1]), b1(p["ln_b"][i + 1]),
            p["Wr"][i + 1], b1(p["br"][i + 1]))

    agg = agg_kernel(hn4.reshape(NQ * N_PAD, QD), src_p, dst4)
    i = NUM_LAYERS - 1
    out = _fin_call(agg.reshape(NQ, N_PAD, QD), cnt, p["Wl"][i],
                    b1(p["bl"][i]), self_t, h, jk, p["Wout"], b1(p["bout"]))
    return out[:N]


# speedup vs baseline: 2.0428x; 2.0428x over previous
import functools

import jax
import jax.numpy as jnp
from jax import lax
from jax.experimental import pallas as pl
from jax.experimental.pallas import tpu as pltpu
from jax.experimental.pallas import tpu_sc as plsc

N = 10000
E = 160000
IN_DIM = 256
HID = 512
OUT_DIM = 128
NUM_LAYERS = 4

N_PAD = 10240
TR = 512
GRID = N_PAD // TR

NQ = 4
QD = HID // NQ
NT = 16
EPT = E // NT
CH = 80
NCH = EPT // CH
RPT = N_PAD // NT
CNT_W = 128
EPT_C = E // (2 * NT)
NCHP = EPT_C // CH + 1


def _count_body(dst2_hbm, cnt_hbm, dst_all, ones_v, acc_sh, ssem):
    c = lax.axis_index("c")
    s = lax.axis_index("s")
    w = c * NT + s
    zeros16 = jnp.zeros((16,), jnp.float32)

    pltpu.sync_copy(dst2_hbm.at[w], dst_all)

    def _zero_row(i, _):
        for j in range(CNT_W // 16):
            ones_v[i, pl.ds(16 * j, 16)] = zeros16
        return 0

    lax.fori_loop(0, CH, _zero_row, 0)
    for k in range(RPT // CH):
        pltpu.sync_copy(ones_v, acc_sh.at[pl.ds(s * RPT + k * CH, CH)])

    def _one_row(i, _):
        for j in range(CNT_W // 16):
            ones_v[i, pl.ds(16 * j, 16)] = zeros16 + 1.0
        return 0

    lax.fori_loop(0, CH, _one_row, 0)
    plsc.subcore_barrier()

    GK = 2

    def _grp(g, _):
        for b in range(GK):
            pltpu.async_copy(ones_v, acc_sh.at[dst_all.at[GK * g + b]],
                             ssem, add=True)
        for b in range(GK):
            pltpu.make_async_copy(ones_v, acc_sh.at[pl.ds(0, CH)],
                                  ssem).wait()
        return 0

    lax.fori_loop(0, NCHP // GK, _grp, 0)
    pltpu.async_copy(ones_v, acc_sh.at[dst_all.at[NCHP - 1]], ssem, add=True)
    pltpu.make_async_copy(ones_v, acc_sh.at[pl.ds(0, CH)], ssem).wait()
    plsc.subcore_barrier()

    pltpu.sync_copy(acc_sh.at[pl.ds(s * RPT, RPT)],
                    cnt_hbm.at[pl.ds(c * N_PAD + s * RPT, RPT)])


def _agg_body(hn_hbm, src_hbm, dst2_hbm, out_hbm, src_flat, dst_all,
              buf0, buf1, acc_sh, sem0, sem1):
    c = lax.axis_index("c")
    s = lax.axis_index("s")
    zeros16 = jnp.zeros((16,), jnp.float32)
    NPAIR = NCH // 2

    pltpu.sync_copy(src_hbm.at[pl.ds(s * EPT, EPT)], src_flat)
    pltpu.sync_copy(dst2_hbm.at[s], dst_all)

    def _gather(j, buf, sem):
        return pltpu.async_copy(hn_hbm.at[src_flat.at[pl.ds(j * CH, CH)]],
                                buf, sem)

    def _wait(buf, sem):
        pltpu.make_async_copy(hn_hbm.at[pl.ds(0, CH)], buf, sem).wait()

    for qi in range(2):
        qoff = (2 * c * N_PAD) if qi == 0 else N_PAD

        def _adjust(i, _):
            src_flat[pl.ds(16 * i, 16)] = src_flat[pl.ds(16 * i, 16)] + qoff
            return 0

        lax.fori_loop(0, EPT // 16, _adjust, 0)

        def _zero_row(i, _):
            for j in range(QD // 16):
                buf0[i, pl.ds(16 * j, 16)] = zeros16
            return 0

        lax.fori_loop(0, CH, _zero_row, 0)
        for k in range(RPT // CH):
            pltpu.sync_copy(buf0, acc_sh.at[pl.ds(s * RPT + k * CH, CH)])
        plsc.subcore_barrier()

        _gather(0, buf0, sem0)

        def _pair(i, _):
            _gather(2 * i + 1, buf1, sem1)
            _wait(buf0, sem0)
            pltpu.sync_copy(buf0, acc_sh.at[dst_all.at[2 * i]], add=True)
            _gather(2 * i + 2, buf0, sem0)
            _wait(buf1, sem1)
            pltpu.sync_copy(buf1, acc_sh.at[dst_all.at[2 * i + 1]], add=True)
            return 0

        lax.fori_loop(0, NPAIR, _pair, 0)
        _wait(buf0, sem0)
        pltpu.sync_copy(buf0, acc_sh.at[dst_all.at[NCH - 1]], add=True)
        plsc.subcore_barrier()

        pltpu.sync_copy(acc_sh.at[pl.ds(s * RPT, RPT)],
                        out_hbm.at[pl.ds((2 * c + qi) * N_PAD + s * RPT, RPT)])


@functools.cache
def _sc_kernels():
    mesh = plsc.VectorSubcoreMesh(core_axis_name="c", subcore_axis_name="s")
    count_kernel = functools.partial(
        pl.kernel,
        out_type=jax.ShapeDtypeStruct((2 * N_PAD, CNT_W), jnp.float32),
        mesh=mesh,
        scratch_types=[
            pltpu.VMEM((NCHP, CH), jnp.int32),
            pltpu.VMEM((CH, CNT_W), jnp.float32),
            pltpu.VMEM_SHARED((N_PAD, CNT_W), jnp.float32),
            pltpu.SemaphoreType.DMA,
        ],
    )(_count_body)
    agg_kernel = functools.partial(
        pl.kernel,
        out_type=jax.ShapeDtypeStruct((NQ * N_PAD, QD), jnp.float32),
        mesh=mesh,
        scratch_types=[
            pltpu.VMEM((EPT,), jnp.int32),
            pltpu.VMEM((NCH, CH), jnp.int32),
            pltpu.VMEM((CH, QD), jnp.float32),
            pltpu.VMEM((CH, QD), jnp.float32),
            pltpu.VMEM_SHARED((N_PAD, QD), jnp.float32),
            pltpu.SemaphoreType.DMA,
            pltpu.SemaphoreType.DMA,
        ],
    )(_agg_body)
    return count_kernel, agg_kernel


def _ln_self(h, g, bt, wr, br):
    mu = jnp.mean(h, axis=1, keepdims=True)
    var = jnp.mean((h - mu) ** 2, axis=1, keepdims=True)
    hn = (h - mu) * lax.rsqrt(var + 1e-5) * g + bt
    self_t = jnp.dot(hn, wr, preferred_element_type=jnp.float32) + br
    return hn.reshape(TR, NQ, QD).transpose(1, 0, 2), self_t


def _mix(agg4, cnt2, wl, bl, self_t, h, jk):
    agg = agg4.transpose(1, 0, 2).reshape(TR, HID)
    scale = 1.0 / jnp.maximum(cnt2[0, :, :1] + cnt2[1, :, :1], 1.0)
    z = (jnp.dot(agg * scale, wl, preferred_element_type=jnp.float32)
         + bl + self_t)
    hnew = jnp.maximum(h + z, 0.0)
    return hnew, jnp.maximum(jk, hnew)


def _in_body(x_ref, win_ref, bin_ref, g_ref, bt_ref, wr_ref, br_ref,
             h_ref, hn_ref, self_ref):
    h = (jnp.dot(x_ref[...], win_ref[...], preferred_element_type=jnp.float32)
         + bin_ref[...])
    h_ref[...] = h
    hn_ref[...], self_ref[...] = _ln_self(h, g_ref[...], bt_ref[...],
                                          wr_ref[...], br_ref[...])


def _layer_body(agg_ref, cnt_ref, wl_ref, bl_ref, self_ref, h_ref, jk_ref,
                g_ref, bt_ref, wr_ref, br_ref,
                ho_ref, jko_ref, hn_ref, selfo_ref):
    hnew, jknew = _mix(agg_ref[...], cnt_ref[...], wl_ref[...], bl_ref[...],
                       self_ref[...], h_ref[...], jk_ref[...])
    ho_ref[...] = hnew
    jko_ref[...] = jknew
    hn_ref[...], selfo_ref[...] = _ln_self(hnew, g_ref[...], bt_ref[...],
                                           wr_ref[...], br_ref[...])


def _fin_body(agg_ref, cnt_ref, wl_ref, bl_ref, self_ref, h_ref, jk_ref,
              wout_ref, bout_ref, o_ref):
    _, jknew = _mix(agg_ref[...], cnt_ref[...], wl_ref[...], bl_ref[...],
                    self_ref[...], h_ref[...], jk_ref[...])
    o_ref[...] = (jnp.dot(jknew, wout_ref[...],
                          preferred_element_type=jnp.float32) + bout_ref[...])


def _row_spec(w):
    return pl.BlockSpec((TR, w), lambda i: (i, 0))


def _full_spec(shape):
    return pl.BlockSpec(shape, lambda i: tuple(0 for _ in shape))


_agg_spec = pl.BlockSpec((NQ, TR, QD), lambda i: (0, i, 0))
_cnt_spec = pl.BlockSpec((2, TR, CNT_W), lambda i: (0, i, 0))
_hn_out = jax.ShapeDtypeStruct((NQ, N_PAD, QD), jnp.float32)
_row_out = jax.ShapeDtypeStruct((N_PAD, HID), jnp.float32)

_in_call = pl.pallas_call(
    _in_body,
    grid=(GRID,),
    in_specs=[_row_spec(IN_DIM), _full_spec((IN_DIM, HID)),
              _full_spec((1, HID)), _full_spec((1, HID)), _full_spec((1, HID)),
              _full_spec((HID, HID)), _full_spec((1, HID))],
    out_specs=[_row_spec(HID), _agg_spec, _row_spec(HID)],
    out_shape=[_row_out, _hn_out, _row_out],
)

_layer_call = pl.pallas_call(
    _layer_body,
    grid=(GRID,),
    in_specs=[_agg_spec, _cnt_spec, _full_spec((HID, HID)),
              _full_spec((1, HID)), _row_spec(HID), _row_spec(HID),
              _row_spec(HID), _full_spec((1, HID)), _full_spec((1, HID)),
              _full_spec((HID, HID)), _full_spec((1, HID))],
    out_specs=[_row_spec(HID), _row_spec(HID), _agg_spec, _row_spec(HID)],
    out_shape=[_row_out, _row_out, _hn_out, _row_out],
)

_fin_call = pl.pallas_call(
    _fin_body,
    grid=(GRID,),
    in_specs=[_agg_spec, _cnt_spec, _full_spec((HID, HID)),
              _full_spec((1, HID)), _row_spec(HID), _row_spec(HID),
              _row_spec(HID), _full_spec((HID, OUT_DIM)),
              _full_spec((1, OUT_DIM))],
    out_specs=_row_spec(OUT_DIM),
    out_shape=jax.ShapeDtypeStruct((N_PAD, OUT_DIM), jnp.float32),
)


def kernel(x, edge_index, params):
    src = edge_index[0].astype(jnp.int32)
    dst = edge_index[1].astype(jnp.int32)
    dst3 = dst.reshape(NT, NCH, CH)
    dstc = jnp.pad(dst.reshape(2 * NT, EPT_C),
                   ((0, 0), (0, NCHP * CH - EPT_C)),
                   constant_values=N_PAD - 1).reshape(2 * NT, NCHP, CH)
    x_p = jnp.pad(x, ((0, N_PAD - N), (0, 0)))

    p = params
    count_kernel, agg_kernel = _sc_kernels()
    cnt = count_kernel(dstc).reshape(2, N_PAD, CNT_W)

    def b1(v):
        return v.reshape(1, -1)

    h, hn4, self_t = _in_call(x_p, p["Win"], b1(p["bin"]), b1(p["ln_g"][0]),
                              b1(p["ln_b"][0]), p["Wr"][0], b1(p["br"][0]))

    jk = jnp.zeros((N_PAD, HID), jnp.float32)
    for i in range(NUM_LAYERS - 1):
        agg = agg_kernel(hn4.reshape(NQ * N_PAD, QD), src, dst3)
        h, jk, hn4, self_t = _layer_call(
            agg.reshape(NQ, N_PAD, QD), cnt, p["Wl"][i], b1(p["bl"][i]),
            self_t, h, jk, b1(p["ln_g"][i + 1]), b1(p["ln_b"][i + 1]),
            p["Wr"][i + 1], b1(p["br"][i + 1]))

    agg = agg_kernel(hn4.reshape(NQ * N_PAD, QD), src, dst3)
    i = NUM_LAYERS - 1
    out = _fin_call(agg.reshape(NQ, N_PAD, QD), cnt, p["Wl"][i],
                    b1(p["bl"][i]), self_t, h, jk, p["Wout"], b1(p["bout"]))
    return out[:N]
